# Initial kernel scaffold; baseline (speedup 1.0000x reference)
import jax
import jax.numpy as jnp
from jax.experimental import pallas as pl


def kernel(x0, W, b, bank):
    raise NotImplementedError("write your pallas kernel here")


# speedup vs baseline: 1.2528x; 1.2528x over previous
import functools

import jax
import jax.numpy as jnp
from jax import lax
from jax.experimental import pallas as pl
from jax.experimental.pallas import tpu as pltpu
from jax.experimental.pallas import tpu_sc as plsc

B, D, K = 4096, 512, 65536
KB = 512
NK = K // KB


def _sim_argmax_kernel(x0_ref, w_ref, b_ref, bank_ref, idx_ref,
                       outn_ref, rmax_ref, ridx_ref):
    j = pl.program_id(0)

    @pl.when(j == 0)
    def _init():
        out0 = jnp.dot(x0_ref[...], w_ref[...]) + b_ref[...]
        nrm = jnp.sqrt(jnp.sum(out0 * out0, axis=1, keepdims=True))
        outn_ref[...] = out0 / (nrm + 1e-12)
        rmax_ref[...] = jnp.full((B, 1), -jnp.inf, jnp.float32)
        ridx_ref[...] = jnp.zeros((B, 1), jnp.int32)

    blk = bank_ref[...]
    bnrm = jnp.sqrt(jnp.sum(blk * blk, axis=1, keepdims=True))
    blk_n = blk / (bnrm + 1e-12)
    scores = lax.dot_general(outn_ref[...], blk_n,
                             (((1,), (1,)), ((), ())))
    bmax = jnp.max(scores, axis=1, keepdims=True)
    iota = lax.broadcasted_iota(jnp.int32, scores.shape, 1)
    lidx = jnp.min(jnp.where(scores == bmax, iota, K),
                   axis=1, keepdims=True)
    gidx = j * KB + lidx
    upd = bmax > rmax_ref[...]
    rmax_ref[...] = jnp.where(upd, bmax, rmax_ref[...])
    ridx_ref[...] = jnp.where(upd, gidx, ridx_ref[...])

    @pl.when(j == NK - 1)
    def _fin():
        idx_ref[...] = ridx_ref[...]


def _nn_indices(x0, W, b2, bank):
    return pl.pallas_call(
        _sim_argmax_kernel,
        grid=(NK,),
        in_specs=[
            pl.BlockSpec((B, D), lambda j: (0, 0)),
            pl.BlockSpec((D, D), lambda j: (0, 0)),
            pl.BlockSpec((1, D), lambda j: (0, 0)),
            pl.BlockSpec((KB, D), lambda j: (j, 0)),
        ],
        out_specs=pl.BlockSpec((B, 1), lambda j: (0, 0)),
        out_shape=jax.ShapeDtypeStruct((B, 1), jnp.int32),
        scratch_shapes=[
            pltpu.VMEM((B, D), jnp.float32),
            pltpu.VMEM((B, 1), jnp.float32),
            pltpu.VMEM((B, 1), jnp.int32),
        ],
    )(x0, W, b2, bank)


_SC_INFO = plsc.get_sparse_core_info()
_NW = _SC_INFO.num_cores * _SC_INFO.num_subcores
_BPW = B // _NW


@functools.partial(
    pl.kernel,
    mesh=plsc.VectorSubcoreMesh(core_axis_name="c", subcore_axis_name="s"),
    out_type=jax.ShapeDtypeStruct((B, D), jnp.float32),
    scratch_types=[
        pltpu.VMEM((_BPW,), jnp.int32),
        pltpu.VMEM((_BPW, D), jnp.float32),
        pltpu.SemaphoreType.DMA,
    ],
)
def _sc_gather(bank_hbm, idx_hbm, out_hbm, idx_v, rows_v, sem):
    wid = lax.axis_index("s") * _SC_INFO.num_cores + lax.axis_index("c")
    base = wid * _BPW
    pltpu.sync_copy(idx_hbm.at[pl.ds(base, _BPW)], idx_v)
    pltpu.async_copy(bank_hbm.at[idx_v], rows_v, sem).wait()
    pltpu.sync_copy(rows_v, out_hbm.at[pl.ds(base, _BPW)])


def kernel(x0, W, b, bank):
    idx = _nn_indices(x0, W, b.reshape(1, D), bank).reshape(B)
    return _sc_gather(bank, idx)


# speedup vs baseline: 1.4710x; 1.1742x over previous
import functools

import jax
import jax.numpy as jnp
from jax import lax
from jax.experimental import pallas as pl
from jax.experimental.pallas import tpu as pltpu
from jax.experimental.pallas import tpu_sc as plsc

B, D, K = 4096, 512, 65536
KB = 512
NK = K // KB


def _mm_into(bank_ref, outn_ref, dst_ref):
    blk = bank_ref[...]
    bnrm = jnp.sqrt(jnp.sum(blk * blk, axis=1, keepdims=True))
    blk_n = blk / (bnrm + 1e-12)
    dst_ref[...] = lax.dot_general(
        outn_ref[...], blk_n, (((1,), (1,)), ((), ())))


def _reduce_from(src_ref, rmax_ref, ridx_ref, j):
    scores = src_ref[...]
    bmax = jnp.max(scores, axis=1, keepdims=True)
    iota = lax.broadcasted_iota(jnp.int32, scores.shape, 1)
    lidx = jnp.min(jnp.where(scores == bmax, iota, K),
                   axis=1, keepdims=True)
    gidx = (j - 1) * KB + lidx
    upd = (bmax > rmax_ref[...]) & (j > 0)
    rmax_ref[...] = jnp.where(upd, bmax, rmax_ref[...])
    ridx_ref[...] = jnp.where(upd, gidx, ridx_ref[...])


def _sim_argmax_kernel(x0_ref, w_ref, b_ref, bank_ref, idx_ref,
                       outn_ref, sc0_ref, sc1_ref, rmax_ref, ridx_ref):
    j = pl.program_id(0)

    @pl.when(j == 0)
    def _init():
        out0 = jnp.dot(x0_ref[...], w_ref[...]) + b_ref[...]
        nrm = jnp.sqrt(jnp.sum(out0 * out0, axis=1, keepdims=True))
        outn_ref[...] = out0 / (nrm + 1e-12)
        rmax_ref[...] = jnp.full((B, 1), -jnp.inf, jnp.float32)
        ridx_ref[...] = jnp.zeros((B, 1), jnp.int32)

    @pl.when(jax.lax.rem(j, 2) == 0)
    def _even():
        _mm_into(bank_ref, outn_ref, sc0_ref)
        _reduce_from(sc1_ref, rmax_ref, ridx_ref, j)

    @pl.when(jax.lax.rem(j, 2) == 1)
    def _odd():
        _mm_into(bank_ref, outn_ref, sc1_ref)
        _reduce_from(sc0_ref, rmax_ref, ridx_ref, j)

    @pl.when(j == NK)
    def _fin():
        idx_ref[...] = ridx_ref[...]


def _nn_indices(x0, W, b2, bank):
    return pl.pallas_call(
        _sim_argmax_kernel,
        grid=(NK + 1,),
        in_specs=[
            pl.BlockSpec((B, D), lambda j: (0, 0)),
            pl.BlockSpec((D, D), lambda j: (0, 0)),
            pl.BlockSpec((1, D), lambda j: (0, 0)),
            pl.BlockSpec((KB, D), lambda j: (jnp.minimum(j, NK - 1), 0)),
        ],
        out_specs=pl.BlockSpec((B, 1), lambda j: (0, 0)),
        out_shape=jax.ShapeDtypeStruct((B, 1), jnp.int32),
        scratch_shapes=[
            pltpu.VMEM((B, D), jnp.float32),
            pltpu.VMEM((B, KB), jnp.float32),
            pltpu.VMEM((B, KB), jnp.float32),
            pltpu.VMEM((B, 1), jnp.float32),
            pltpu.VMEM((B, 1), jnp.int32),
        ],
    )(x0, W, b2, bank)


_SC_INFO = plsc.get_sparse_core_info()
_NW = _SC_INFO.num_cores * _SC_INFO.num_subcores
_BPW = B // _NW


@functools.partial(
    pl.kernel,
    mesh=plsc.VectorSubcoreMesh(core_axis_name="c", subcore_axis_name="s"),
    out_type=jax.ShapeDtypeStruct((B, D), jnp.float32),
    scratch_types=[
        pltpu.VMEM((_BPW,), jnp.int32),
        pltpu.VMEM((_BPW, D), jnp.float32),
        pltpu.SemaphoreType.DMA,
    ],
)
def _sc_gather(bank_hbm, idx_hbm, out_hbm, idx_v, rows_v, sem):
    wid = lax.axis_index("s") * _SC_INFO.num_cores + lax.axis_index("c")
    base = wid * _BPW
    pltpu.sync_copy(idx_hbm.at[pl.ds(base, _BPW)], idx_v)
    pltpu.async_copy(bank_hbm.at[idx_v], rows_v, sem).wait()
    pltpu.sync_copy(rows_v, out_hbm.at[pl.ds(base, _BPW)])


def kernel(x0, W, b, bank):
    idx = _nn_indices(x0, W, b.reshape(1, D), bank).reshape(B)
    return _sc_gather(bank, idx)


# speedup vs baseline: 1.6001x; 1.0877x over previous
import functools

import jax
import jax.numpy as jnp
from jax import lax
from jax.experimental import pallas as pl
from jax.experimental.pallas import tpu as pltpu
from jax.experimental.pallas import tpu_sc as plsc

B, D, K = 4096, 512, 65536
KB = 512
NK = K // KB


def _mm_into(bank_ref, outn_ref, dst_ref):
    blk = bank_ref[...]
    bnrm = jnp.sqrt(jnp.sum(blk * blk, axis=1, keepdims=True))
    blk_n = blk / (bnrm + 1e-12)
    dst_ref[...] = lax.dot_general(
        outn_ref[...], blk_n, (((1,), (1,)), ((), ())))


def _reduce_from(src_ref, rmax_ref, ridx_ref, j):
    scores = src_ref[...]
    bmax = jnp.max(scores, axis=1, keepdims=True)
    iota = lax.broadcasted_iota(jnp.int32, scores.shape, 1).astype(jnp.float32)
    lidx = jnp.min(jnp.where(scores == bmax, iota, float(K)),
                   axis=1, keepdims=True)
    gidx = ((j - 1) * KB).astype(jnp.float32) + lidx
    upd = (bmax > rmax_ref[...]) & (j > 0)
    rmax_ref[...] = jnp.where(upd, bmax, rmax_ref[...])
    ridx_ref[...] = jnp.where(upd, gidx, ridx_ref[...])


def _sim_argmax_kernel(x0_ref, w_ref, b_ref, bank_ref, idx_ref,
                       outn_ref, sc0_ref, sc1_ref, rmax_ref, ridx_ref):
    j = pl.program_id(0)

    @pl.when(j == 0)
    def _init():
        out0 = jnp.dot(x0_ref[...], w_ref[...]) + b_ref[...]
        nrm = jnp.sqrt(jnp.sum(out0 * out0, axis=1, keepdims=True))
        outn_ref[...] = out0 / (nrm + 1e-12)
        rmax_ref[...] = jnp.full((B, 1), -jnp.inf, jnp.float32)
        ridx_ref[...] = jnp.zeros((B, 1), jnp.float32)

    @pl.when(jax.lax.rem(j, 2) == 0)
    def _even():
        _mm_into(bank_ref, outn_ref, sc0_ref)
        _reduce_from(sc1_ref, rmax_ref, ridx_ref, j)

    @pl.when(jax.lax.rem(j, 2) == 1)
    def _odd():
        _mm_into(bank_ref, outn_ref, sc1_ref)
        _reduce_from(sc0_ref, rmax_ref, ridx_ref, j)

    @pl.when(j == NK)
    def _fin():
        idx_ref[...] = ridx_ref[...].astype(jnp.int32)


def _nn_indices(x0, W, b2, bank):
    return pl.pallas_call(
        _sim_argmax_kernel,
        grid=(NK + 1,),
        in_specs=[
            pl.BlockSpec((B, D), lambda j: (0, 0)),
            pl.BlockSpec((D, D), lambda j: (0, 0)),
            pl.BlockSpec((1, D), lambda j: (0, 0)),
            pl.BlockSpec((KB, D), lambda j: (jnp.minimum(j, NK - 1), 0)),
        ],
        out_specs=pl.BlockSpec((B, 1), lambda j: (0, 0)),
        out_shape=jax.ShapeDtypeStruct((B, 1), jnp.int32),
        scratch_shapes=[
            pltpu.VMEM((B, D), jnp.float32),
            pltpu.VMEM((B, KB), jnp.float32),
            pltpu.VMEM((B, KB), jnp.float32),
            pltpu.VMEM((B, 1), jnp.float32),
            pltpu.VMEM((B, 1), jnp.float32),
        ],
    )(x0, W, b2, bank)


_SC_INFO = plsc.get_sparse_core_info()
_NW = _SC_INFO.num_cores * _SC_INFO.num_subcores
_BPW = B // _NW


@functools.partial(
    pl.kernel,
    mesh=plsc.VectorSubcoreMesh(core_axis_name="c", subcore_axis_name="s"),
    out_type=jax.ShapeDtypeStruct((B, D), jnp.float32),
    scratch_types=[
        pltpu.VMEM((_BPW,), jnp.int32),
        pltpu.VMEM((_BPW, D), jnp.float32),
        pltpu.SemaphoreType.DMA,
    ],
)
def _sc_gather(bank_hbm, idx_hbm, out_hbm, idx_v, rows_v, sem):
    wid = lax.axis_index("s") * _SC_INFO.num_cores + lax.axis_index("c")
    base = wid * _BPW
    pltpu.sync_copy(idx_hbm.at[pl.ds(base, _BPW)], idx_v)
    pltpu.async_copy(bank_hbm.at[idx_v], rows_v, sem).wait()
    pltpu.sync_copy(rows_v, out_hbm.at[pl.ds(base, _BPW)])


def kernel(x0, W, b, bank):
    idx = _nn_indices(x0, W, b.reshape(1, D), bank).reshape(B)
    return _sc_gather(bank, idx)
